# Initial kernel scaffold; baseline (speedup 1.0000x reference)
import jax
import jax.numpy as jnp
from jax.experimental import pallas as pl


def kernel(features, tois, padding):
    raise NotImplementedError("write your pallas kernel here")


# speedup vs baseline: 3.6941x; 3.6941x over previous
import functools

import numpy as np
import jax
import jax.numpy as jnp
from jax import lax
from jax.experimental import pallas as pl
from jax.experimental.pallas import tpu as pltpu
from jax.experimental.pallas import tpu_sc as plsc

_B, _D, _S, _R = 8, 512, 2048, 128
_ROWS = _S + 8
_C = 256
_NW = 32
_NREG = _B * _R
_BPW = _NREG // _NW
_LANES = 16


def _cumsum_tc_kernel(f_ref, pad_ref, out_ref):
    pad_row = pad_ref[0].reshape(1, _D)
    out_ref[0, 0:8, :] = jnp.broadcast_to(pad_row, (8, _D))
    row = lax.broadcasted_iota(jnp.int32, (_C, _C), 0)
    col = lax.broadcasted_iota(jnp.int32, (_C, _C), 1)
    ltri = (col <= row).astype(jnp.float32)
    carry = pad_row
    for k in range(_S // _C):
        fc = f_ref[0, :, k * _C:(k + 1) * _C]
        ic = lax.dot_general(
            ltri, fc,
            dimension_numbers=(((1,), (1,)), ((), ())),
            preferred_element_type=jnp.float32,
            precision=lax.Precision.HIGHEST,
        )
        ic = ic + carry
        out_ref[0, 8 + k * _C: 8 + (k + 1) * _C, :] = ic
        carry = ic[_C - 1:_C, :]


_sc_mesh = plsc.VectorSubcoreMesh(core_axis_name="c", subcore_axis_name="s")


@functools.partial(
    pl.kernel,
    out_type=jax.ShapeDtypeStruct((_NREG, 3 * _D), jnp.float32),
    mesh=_sc_mesh,
    scratch_types=[
        pltpu.VMEM((_BPW,), jnp.int32),
        pltpu.VMEM((_BPW,), jnp.int32),
        pltpu.VMEM((_BPW,), jnp.int32),
        pltpu.VMEM((_BPW,), jnp.int32),
        pltpu.VMEM((_BPW, _LANES), jnp.float32),
        pltpu.VMEM((_BPW, _D), jnp.float32),
        pltpu.VMEM((_BPW, _D), jnp.float32),
        pltpu.VMEM((_BPW, _D), jnp.float32),
        pltpu.VMEM((_BPW, _D), jnp.float32),
        pltpu.VMEM((_BPW, 3 * _D), jnp.float32),
        pltpu.SemaphoreType.DMA,
        pltpu.SemaphoreType.DMA,
        pltpu.SemaphoreType.DMA,
        pltpu.SemaphoreType.DMA,
    ],
)
def _region_sc_kernel(table_hbm, i1h, i2h, i3h, i4h, invh, out_hbm,
                      i1v, i2v, i3v, i4v, invv, g1, g2, g3, g4, outv,
                      s1, s2, s3, s4):
    wid = lax.axis_index("s") * 2 + lax.axis_index("c")
    base = wid * _BPW
    pltpu.sync_copy(i1h.at[pl.ds(base, _BPW)], i1v)
    pltpu.sync_copy(i2h.at[pl.ds(base, _BPW)], i2v)
    pltpu.sync_copy(i3h.at[pl.ds(base, _BPW)], i3v)
    pltpu.sync_copy(i4h.at[pl.ds(base, _BPW)], i4v)
    pltpu.sync_copy(invh.at[pl.ds(base, _BPW), :], invv)
    c1 = pltpu.async_copy(table_hbm.at[i1v], g1, s1)
    c2 = pltpu.async_copy(table_hbm.at[i2v], g2, s2)
    c3 = pltpu.async_copy(table_hbm.at[i3v], g3, s3)
    c4 = pltpu.async_copy(table_hbm.at[i4v], g4, s4)
    c1.wait()
    c2.wait()
    c3.wait()
    c4.wait()

    def region_body(i, carry):
        ilv = invv[i, :]

        def lane_body(cidx, carry2):
            off = cidx * _LANES
            a1 = g1[i, pl.ds(off, _LANES)]
            a2 = g2[i, pl.ds(off, _LANES)]
            a3 = g3[i, pl.ds(off, _LANES)]
            a4 = g4[i, pl.ds(off, _LANES)]
            outv[i, pl.ds(off, _LANES)] = a2 - a1
            outv[i, pl.ds(_D + off, _LANES)] = (a4 - a1) * ilv
            outv[i, pl.ds(2 * _D + off, _LANES)] = a4 - a3
            return carry2

        lax.fori_loop(0, _D // _LANES, lane_body, 0)
        return carry

    lax.fori_loop(0, _BPW, region_body, 0)
    pltpu.sync_copy(outv, out_hbm.at[pl.ds(base, _BPW)])


def kernel(features, tois, padding):
    cs = pl.pallas_call(
        _cumsum_tc_kernel,
        grid=(_B,),
        in_specs=[
            pl.BlockSpec((1, _D, _S), lambda b: (b, 0, 0)),
            pl.BlockSpec((1, _D, 1), lambda b: (0, 0, 0)),
        ],
        out_specs=pl.BlockSpec((1, _ROWS, _D), lambda b: (b, 0, 0)),
        out_shape=jax.ShapeDtypeStruct((_B, _ROWS, _D), jnp.float32),
    )(features, padding.reshape(1, _D, 1))
    table = cs.reshape(_B * _ROWS, _D)

    tois32 = tois.astype(jnp.int32)
    s = tois32[..., 0].reshape(-1)
    e = tois32[..., 1].reshape(-1)
    boff = jnp.repeat(jnp.arange(_B, dtype=jnp.int32) * _ROWS, _R)
    i1 = boff + 7 + s
    i2 = i1 + 1
    i3 = boff + 6 + e
    i4 = boff + 7 + e
    invlen = 1.0 / (e - s).astype(jnp.float32)
    invb = jnp.broadcast_to(invlen[:, None], (_NREG, _LANES))

    result = _region_sc_kernel(table, i1, i2, i3, i4, invb)
    lengths = jnp.asarray(np.cumsum([_R] * _B).astype(np.int32))
    return (result, lengths)


# speedup vs baseline: 4.4058x; 1.1927x over previous
import functools

import numpy as np
import jax
import jax.numpy as jnp
from jax import lax
from jax.experimental import pallas as pl
from jax.experimental.pallas import tpu as pltpu
from jax.experimental.pallas import tpu_sc as plsc

_B, _D, _S, _R = 8, 512, 2048, 128
_ROWS = _S + 8
_C = 256
_NW = 32
_NREG = _B * _R
_BPW = _NREG // _NW
_LANES = 16


def _cumsum_tc_kernel(f_ref, pad_ref, out_ref):
    pad_row = pad_ref[0].reshape(1, _D)
    out_ref[0, 0:8, :] = jnp.broadcast_to(pad_row, (8, _D))
    row = lax.broadcasted_iota(jnp.int32, (_C, _C), 0)
    col = lax.broadcasted_iota(jnp.int32, (_C, _C), 1)
    ltri = (col <= row).astype(jnp.float32)
    carry = pad_row
    for k in range(_S // _C):
        fc = f_ref[0, :, k * _C:(k + 1) * _C]
        ic = lax.dot_general(
            ltri, fc,
            dimension_numbers=(((1,), (1,)), ((), ())),
            preferred_element_type=jnp.float32,
        )
        ic = ic + carry
        out_ref[0, 8 + k * _C: 8 + (k + 1) * _C, :] = ic
        carry = ic[_C - 1:_C, :]


_sc_mesh = plsc.VectorSubcoreMesh(core_axis_name="c", subcore_axis_name="s")


@functools.partial(
    pl.kernel,
    out_type=jax.ShapeDtypeStruct((_NREG, 3 * _D), jnp.float32),
    mesh=_sc_mesh,
    scratch_types=[
        pltpu.VMEM((_BPW,), jnp.int32),
        pltpu.VMEM((_BPW,), jnp.int32),
        pltpu.VMEM((_BPW,), jnp.int32),
        pltpu.VMEM((_BPW,), jnp.int32),
        pltpu.VMEM((_BPW, _LANES), jnp.float32),
        pltpu.VMEM((_BPW, _D), jnp.float32),
        pltpu.VMEM((_BPW, _D), jnp.float32),
        pltpu.VMEM((_BPW, _D), jnp.float32),
        pltpu.VMEM((_BPW, _D), jnp.float32),
        pltpu.VMEM((_BPW, 3 * _D), jnp.float32),
        pltpu.SemaphoreType.DMA,
        pltpu.SemaphoreType.DMA,
        pltpu.SemaphoreType.DMA,
        pltpu.SemaphoreType.DMA,
    ],
)
def _region_sc_kernel(table_hbm, i1h, i2h, i3h, i4h, invh, out_hbm,
                      i1v, i2v, i3v, i4v, invv, g1, g2, g3, g4, outv,
                      s1, s2, s3, s4):
    wid = lax.axis_index("s") * 2 + lax.axis_index("c")
    base = wid * _BPW
    pltpu.sync_copy(i1h.at[pl.ds(base, _BPW)], i1v)
    pltpu.sync_copy(i2h.at[pl.ds(base, _BPW)], i2v)
    pltpu.sync_copy(i3h.at[pl.ds(base, _BPW)], i3v)
    pltpu.sync_copy(i4h.at[pl.ds(base, _BPW)], i4v)
    pltpu.sync_copy(invh.at[pl.ds(base, _BPW), :], invv)
    c1 = pltpu.async_copy(table_hbm.at[i1v], g1, s1)
    c2 = pltpu.async_copy(table_hbm.at[i2v], g2, s2)
    c3 = pltpu.async_copy(table_hbm.at[i3v], g3, s3)
    c4 = pltpu.async_copy(table_hbm.at[i4v], g4, s4)
    c1.wait()
    c2.wait()
    c3.wait()
    c4.wait()

    _UNROLL = 4

    def region_body(i, carry):
        ilv = invv[i, :]

        def lane_body(cidx, carry2):
            for u in range(_UNROLL):
                off = (cidx * _UNROLL + u) * _LANES
                a1 = g1[i, pl.ds(off, _LANES)]
                a2 = g2[i, pl.ds(off, _LANES)]
                a3 = g3[i, pl.ds(off, _LANES)]
                a4 = g4[i, pl.ds(off, _LANES)]
                outv[i, pl.ds(off, _LANES)] = a2 - a1
                outv[i, pl.ds(_D + off, _LANES)] = (a4 - a1) * ilv
                outv[i, pl.ds(2 * _D + off, _LANES)] = a4 - a3
            return carry2

        lax.fori_loop(0, _D // _LANES // _UNROLL, lane_body, 0)
        return carry

    lax.fori_loop(0, _BPW, region_body, 0)
    pltpu.sync_copy(outv, out_hbm.at[pl.ds(base, _BPW)])


def kernel(features, tois, padding):
    cs = pl.pallas_call(
        _cumsum_tc_kernel,
        grid=(_B,),
        in_specs=[
            pl.BlockSpec((1, _D, _S), lambda b: (b, 0, 0)),
            pl.BlockSpec((1, _D, 1), lambda b: (0, 0, 0)),
        ],
        out_specs=pl.BlockSpec((1, _ROWS, _D), lambda b: (b, 0, 0)),
        out_shape=jax.ShapeDtypeStruct((_B, _ROWS, _D), jnp.float32),
    )(features, padding.reshape(1, _D, 1))
    table = cs.reshape(_B * _ROWS, _D)

    tois32 = tois.astype(jnp.int32)
    s = tois32[..., 0].reshape(-1)
    e = tois32[..., 1].reshape(-1)
    boff = jnp.repeat(jnp.arange(_B, dtype=jnp.int32) * _ROWS, _R)
    i1 = boff + 7 + s
    i2 = i1 + 1
    i3 = boff + 6 + e
    i4 = boff + 7 + e
    invlen = 1.0 / (e - s).astype(jnp.float32)
    invb = jnp.broadcast_to(invlen[:, None], (_NREG, _LANES))

    result = _region_sc_kernel(table, i1, i2, i3, i4, invb)
    lengths = jnp.asarray(np.cumsum([_R] * _B).astype(np.int32))
    return (result, lengths)


# speedup vs baseline: 4.4087x; 1.0007x over previous
import functools

import numpy as np
import jax
import jax.numpy as jnp
from jax import lax
from jax.experimental import pallas as pl
from jax.experimental.pallas import tpu as pltpu
from jax.experimental.pallas import tpu_sc as plsc

_B, _D, _S, _R = 8, 512, 2048, 128
_ROWS = _S + 8
_C = 256
_NW = 32
_NREG = _B * _R
_BPW = _NREG // _NW
_LANES = 16


def _cumsum_tc_kernel(f_ref, pad_ref, out_ref):
    pad_row = pad_ref[0].reshape(1, _D)
    out_ref[0, 0:8, :] = jnp.broadcast_to(pad_row, (8, _D))
    row = lax.broadcasted_iota(jnp.int32, (_C, _C), 0)
    col = lax.broadcasted_iota(jnp.int32, (_C, _C), 1)
    ltri = (col <= row).astype(jnp.bfloat16)
    carry = pad_row
    for k in range(_S // _C):
        fc = f_ref[0, :, k * _C:(k + 1) * _C].astype(jnp.bfloat16)
        ic = lax.dot_general(
            ltri, fc,
            dimension_numbers=(((1,), (1,)), ((), ())),
            preferred_element_type=jnp.float32,
        )
        ic = ic + carry
        out_ref[0, 8 + k * _C: 8 + (k + 1) * _C, :] = ic
        carry = ic[_C - 1:_C, :]


_sc_mesh = plsc.VectorSubcoreMesh(core_axis_name="c", subcore_axis_name="s")


@functools.partial(
    pl.kernel,
    out_type=jax.ShapeDtypeStruct((_NREG, 3 * _D), jnp.float32),
    mesh=_sc_mesh,
    scratch_types=[
        pltpu.VMEM((_BPW,), jnp.int32),
        pltpu.VMEM((_BPW,), jnp.int32),
        pltpu.VMEM((_BPW,), jnp.int32),
        pltpu.VMEM((_BPW,), jnp.int32),
        pltpu.VMEM((_BPW, _LANES), jnp.float32),
        pltpu.VMEM((_BPW, _D), jnp.float32),
        pltpu.VMEM((_BPW, _D), jnp.float32),
        pltpu.VMEM((_BPW, _D), jnp.float32),
        pltpu.VMEM((_BPW, _D), jnp.float32),
        pltpu.VMEM((_BPW, 3 * _D), jnp.float32),
        pltpu.SemaphoreType.DMA,
        pltpu.SemaphoreType.DMA,
        pltpu.SemaphoreType.DMA,
        pltpu.SemaphoreType.DMA,
    ],
)
def _region_sc_kernel(table_hbm, i1h, i2h, i3h, i4h, invh, out_hbm,
                      i1v, i2v, i3v, i4v, invv, g1, g2, g3, g4, outv,
                      s1, s2, s3, s4):
    wid = lax.axis_index("s") * 2 + lax.axis_index("c")
    base = wid * _BPW
    pltpu.sync_copy(i1h.at[pl.ds(base, _BPW)], i1v)
    pltpu.sync_copy(i2h.at[pl.ds(base, _BPW)], i2v)
    pltpu.sync_copy(i3h.at[pl.ds(base, _BPW)], i3v)
    pltpu.sync_copy(i4h.at[pl.ds(base, _BPW)], i4v)
    pltpu.sync_copy(invh.at[pl.ds(base, _BPW), :], invv)
    c1 = pltpu.async_copy(table_hbm.at[i1v], g1, s1)
    c2 = pltpu.async_copy(table_hbm.at[i2v], g2, s2)
    c3 = pltpu.async_copy(table_hbm.at[i3v], g3, s3)
    c4 = pltpu.async_copy(table_hbm.at[i4v], g4, s4)
    c1.wait()
    c2.wait()
    c3.wait()
    c4.wait()

    _UNROLL = 4

    def region_body(i, carry):
        ilv = invv[i, :]

        def lane_body(cidx, carry2):
            for u in range(_UNROLL):
                off = (cidx * _UNROLL + u) * _LANES
                a1 = g1[i, pl.ds(off, _LANES)]
                a2 = g2[i, pl.ds(off, _LANES)]
                a3 = g3[i, pl.ds(off, _LANES)]
                a4 = g4[i, pl.ds(off, _LANES)]
                outv[i, pl.ds(off, _LANES)] = a2 - a1
                outv[i, pl.ds(_D + off, _LANES)] = (a4 - a1) * ilv
                outv[i, pl.ds(2 * _D + off, _LANES)] = a4 - a3
            return carry2

        lax.fori_loop(0, _D // _LANES // _UNROLL, lane_body, 0)
        return carry

    lax.fori_loop(0, _BPW, region_body, 0)
    pltpu.sync_copy(outv, out_hbm.at[pl.ds(base, _BPW)])


def kernel(features, tois, padding):
    cs = pl.pallas_call(
        _cumsum_tc_kernel,
        grid=(_B,),
        in_specs=[
            pl.BlockSpec((1, _D, _S), lambda b: (b, 0, 0)),
            pl.BlockSpec((1, _D, 1), lambda b: (0, 0, 0)),
        ],
        out_specs=pl.BlockSpec((1, _ROWS, _D), lambda b: (b, 0, 0)),
        out_shape=jax.ShapeDtypeStruct((_B, _ROWS, _D), jnp.float32),
    )(features, padding.reshape(1, _D, 1))
    table = cs.reshape(_B * _ROWS, _D)

    tois32 = tois.astype(jnp.int32)
    s = tois32[..., 0].reshape(-1)
    e = tois32[..., 1].reshape(-1)
    boff = jnp.repeat(jnp.arange(_B, dtype=jnp.int32) * _ROWS, _R)
    i1 = boff + 7 + s
    i2 = i1 + 1
    i3 = boff + 6 + e
    i4 = boff + 7 + e
    invlen = 1.0 / (e - s).astype(jnp.float32)
    invb = jnp.broadcast_to(invlen[:, None], (_NREG, _LANES))

    result = _region_sc_kernel(table, i1, i2, i3, i4, invb)
    lengths = jnp.asarray(np.cumsum([_R] * _B).astype(np.int32))
    return (result, lengths)


# speedup vs baseline: 4.4090x; 1.0001x over previous
import functools

import numpy as np
import jax
import jax.numpy as jnp
from jax import lax
from jax.experimental import pallas as pl
from jax.experimental.pallas import tpu as pltpu
from jax.experimental.pallas import tpu_sc as plsc

_B, _D, _S, _R = 8, 512, 2048, 128
_ROWS = _S + 8
_C = 256
_NW = 32
_NREG = _B * _R
_BPW = _NREG // _NW
_LANES = 16


def _cumsum_tc_kernel(f_ref, pad_ref, out_ref):
    pad_row = pad_ref[0].reshape(1, _D)
    out_ref[0, 0:8, :] = jnp.broadcast_to(pad_row, (8, _D))
    row = lax.broadcasted_iota(jnp.int32, (_C, _C), 0)
    col = lax.broadcasted_iota(jnp.int32, (_C, _C), 1)
    ltri = (col <= row).astype(jnp.bfloat16)
    carry = pad_row
    for k in range(_S // _C):
        fc = f_ref[0, :, k * _C:(k + 1) * _C].astype(jnp.bfloat16)
        ic = lax.dot_general(
            ltri, fc,
            dimension_numbers=(((1,), (1,)), ((), ())),
            preferred_element_type=jnp.float32,
        )
        ic = ic + carry
        out_ref[0, 8 + k * _C: 8 + (k + 1) * _C, :] = ic
        carry = ic[_C - 1:_C, :]


_sc_mesh = plsc.VectorSubcoreMesh(core_axis_name="c", subcore_axis_name="s")


@functools.partial(
    pl.kernel,
    out_type=jax.ShapeDtypeStruct((_NREG, 3 * _D), jnp.float32),
    mesh=_sc_mesh,
    scratch_types=[
        pltpu.VMEM((_BPW,), jnp.int32),
        pltpu.VMEM((_BPW,), jnp.int32),
        pltpu.VMEM((_BPW,), jnp.int32),
        pltpu.VMEM((_BPW,), jnp.int32),
        pltpu.VMEM((_BPW, _LANES), jnp.float32),
        pltpu.VMEM((_BPW, _D), jnp.float32),
        pltpu.VMEM((_BPW, _D), jnp.float32),
        pltpu.VMEM((_BPW, _D), jnp.float32),
        pltpu.VMEM((_BPW, _D), jnp.float32),
        pltpu.VMEM((_BPW, 3 * _D), jnp.float32),
        pltpu.SemaphoreType.DMA,
        pltpu.SemaphoreType.DMA,
        pltpu.SemaphoreType.DMA,
        pltpu.SemaphoreType.DMA,
    ],
)
def _region_sc_kernel(table_hbm, i1h, i2h, i3h, i4h, invh, out_hbm,
                      i1v, i2v, i3v, i4v, invv, g1, g2, g3, g4, outv,
                      s1, s2, s3, s4):
    wid = lax.axis_index("s") * 2 + lax.axis_index("c")
    base = wid * _BPW
    pltpu.sync_copy(i1h.at[pl.ds(base, _BPW)], i1v)
    pltpu.sync_copy(i2h.at[pl.ds(base, _BPW)], i2v)
    pltpu.sync_copy(i3h.at[pl.ds(base, _BPW)], i3v)
    pltpu.sync_copy(i4h.at[pl.ds(base, _BPW)], i4v)
    pltpu.sync_copy(invh.at[pl.ds(base, _BPW), :], invv)
    c1 = pltpu.async_copy(table_hbm.at[i1v], g1, s1)
    c2 = pltpu.async_copy(table_hbm.at[i2v], g2, s2)
    c3 = pltpu.async_copy(table_hbm.at[i3v], g3, s3)
    c4 = pltpu.async_copy(table_hbm.at[i4v], g4, s4)
    c1.wait()
    c2.wait()
    c3.wait()
    c4.wait()

    _UNROLL = 4

    def region_body(i, carry):
        ilv = invv[i, :]

        def lane_body(cidx, carry2):
            for u in range(_UNROLL):
                off = (cidx * _UNROLL + u) * _LANES
                a1 = g1[i, pl.ds(off, _LANES)]
                a2 = g2[i, pl.ds(off, _LANES)]
                a3 = g3[i, pl.ds(off, _LANES)]
                a4 = g4[i, pl.ds(off, _LANES)]
                outv[i, pl.ds(off, _LANES)] = a2 - a1
                outv[i, pl.ds(_D + off, _LANES)] = (a4 - a1) * ilv
                outv[i, pl.ds(2 * _D + off, _LANES)] = a4 - a3
            return carry2

        lax.fori_loop(0, _D // _LANES // _UNROLL, lane_body, 0)
        return carry

    lax.fori_loop(0, _BPW, region_body, 0)
    pltpu.sync_copy(outv, out_hbm.at[pl.ds(base, _BPW)])


def kernel(features, tois, padding):
    cs = pl.pallas_call(
        _cumsum_tc_kernel,
        grid=(_B,),
        in_specs=[
            pl.BlockSpec((1, _D, _S), lambda b: (b, 0, 0)),
            pl.BlockSpec((1, _D, 1), lambda b: (0, 0, 0)),
        ],
        out_specs=pl.BlockSpec((1, _ROWS, _D), lambda b: (b, 0, 0)),
        out_shape=jax.ShapeDtypeStruct((_B, _ROWS, _D), jnp.float32),
    )(features, padding.reshape(1, _D, 1))
    table = cs.reshape(_B * _ROWS, _D)

    tois32 = tois.astype(jnp.int32)
    boff2d = jnp.arange(_B, dtype=jnp.int32)[:, None] * _ROWS
    s = (tois32[..., 0] + boff2d).reshape(-1)
    e = (tois32[..., 1] + boff2d).reshape(-1)
    i1 = s + 7
    i2 = s + 8
    i3 = e + 6
    i4 = e + 7
    invlen = 1.0 / (e - s).astype(jnp.float32)
    invb = jnp.broadcast_to(invlen[:, None], (_NREG, _LANES))

    result = _region_sc_kernel(table, i1, i2, i3, i4, invb)
    lengths = jnp.asarray(np.cumsum([_R] * _B).astype(np.int32))
    return (result, lengths)


# speedup vs baseline: 4.6973x; 1.0654x over previous
import functools

import numpy as np
import jax
import jax.numpy as jnp
from jax import lax
from jax.experimental import pallas as pl
from jax.experimental.pallas import tpu as pltpu
from jax.experimental.pallas import tpu_sc as plsc

_B, _D, _S, _R = 8, 512, 2048, 128
_ROWS = _S + 8
_C = 256
_NW = 32
_NREG = _B * _R
_BPW = _NREG // _NW
_LANES = 16


def _cumsum_tc_kernel(f_ref, pad_ref, out_ref):
    pad_row = pad_ref[0].reshape(1, _D)
    out_ref[0, 0:8, :] = jnp.broadcast_to(pad_row, (8, _D))
    row = lax.broadcasted_iota(jnp.int32, (_C, _C), 0)
    col = lax.broadcasted_iota(jnp.int32, (_C, _C), 1)
    ltri = (col <= row).astype(jnp.bfloat16)
    carry = pad_row
    for k in range(_S // _C):
        fc = f_ref[0, :, k * _C:(k + 1) * _C].astype(jnp.bfloat16)
        ic = lax.dot_general(
            ltri, fc,
            dimension_numbers=(((1,), (1,)), ((), ())),
            preferred_element_type=jnp.float32,
        )
        ic = ic + carry
        out_ref[0, 8 + k * _C: 8 + (k + 1) * _C, :] = ic
        carry = ic[_C - 1:_C, :]


_sc_mesh = plsc.VectorSubcoreMesh(core_axis_name="c", subcore_axis_name="s")


@functools.partial(
    pl.kernel,
    out_type=jax.ShapeDtypeStruct((_NREG, 3 * _D), jnp.float32),
    mesh=_sc_mesh,
    scratch_types=[
        pltpu.VMEM((_BPW,), jnp.int32),
        pltpu.VMEM((_BPW,), jnp.int32),
        pltpu.VMEM((_BPW,), jnp.int32),
        pltpu.VMEM((_BPW,), jnp.int32),
        pltpu.VMEM((_BPW, _LANES), jnp.float32),
        pltpu.VMEM((_BPW, _D), jnp.float32),
        pltpu.VMEM((_BPW, _D), jnp.float32),
        pltpu.VMEM((_BPW, _D), jnp.float32),
        pltpu.VMEM((_BPW, _D), jnp.float32),
        pltpu.VMEM((_BPW, 3 * _D), jnp.float32),
        pltpu.SemaphoreType.DMA,
        pltpu.SemaphoreType.DMA,
        pltpu.SemaphoreType.DMA,
        pltpu.SemaphoreType.DMA,
    ],
)
def _region_sc_kernel(table_hbm, i1h, i2h, i3h, i4h, invh, out_hbm,
                      i1v, i2v, i3v, i4v, invv, g1, g2, g3, g4, outv,
                      s1, s2, s3, s4):
    wid = lax.axis_index("s") * 2 + lax.axis_index("c")
    base = wid * _BPW
    pltpu.sync_copy(i1h.at[pl.ds(base, _BPW)], i1v)
    pltpu.sync_copy(i2h.at[pl.ds(base, _BPW)], i2v)
    pltpu.sync_copy(i3h.at[pl.ds(base, _BPW)], i3v)
    pltpu.sync_copy(i4h.at[pl.ds(base, _BPW)], i4v)
    pltpu.sync_copy(invh.at[pl.ds(base, _BPW), :], invv)
    c1 = pltpu.async_copy(table_hbm.at[i1v], g1, s1)
    c2 = pltpu.async_copy(table_hbm.at[i2v], g2, s2)
    c3 = pltpu.async_copy(table_hbm.at[i3v], g3, s3)
    c4 = pltpu.async_copy(table_hbm.at[i4v], g4, s4)
    c1.wait()
    c2.wait()
    c3.wait()
    c4.wait()

    @plsc.parallel_loop(0, _BPW)
    def _region_body(i):
        ilv = invv[i, :]
        for c in range(_D // _LANES):
            off = c * _LANES
            a1 = g1[i, pl.ds(off, _LANES)]
            a2 = g2[i, pl.ds(off, _LANES)]
            a3 = g3[i, pl.ds(off, _LANES)]
            a4 = g4[i, pl.ds(off, _LANES)]
            outv[i, pl.ds(off, _LANES)] = a2 - a1
            outv[i, pl.ds(_D + off, _LANES)] = (a4 - a1) * ilv
            outv[i, pl.ds(2 * _D + off, _LANES)] = a4 - a3
    pltpu.sync_copy(outv, out_hbm.at[pl.ds(base, _BPW)])


def kernel(features, tois, padding):
    cs = pl.pallas_call(
        _cumsum_tc_kernel,
        grid=(_B,),
        in_specs=[
            pl.BlockSpec((1, _D, _S), lambda b: (b, 0, 0)),
            pl.BlockSpec((1, _D, 1), lambda b: (0, 0, 0)),
        ],
        out_specs=pl.BlockSpec((1, _ROWS, _D), lambda b: (b, 0, 0)),
        out_shape=jax.ShapeDtypeStruct((_B, _ROWS, _D), jnp.float32),
    )(features, padding.reshape(1, _D, 1))
    table = cs.reshape(_B * _ROWS, _D)

    tois32 = tois.astype(jnp.int32)
    boff2d = jnp.arange(_B, dtype=jnp.int32)[:, None] * _ROWS
    s = (tois32[..., 0] + boff2d).reshape(-1)
    e = (tois32[..., 1] + boff2d).reshape(-1)
    i1 = s + 7
    i2 = s + 8
    i3 = e + 6
    i4 = e + 7
    invlen = 1.0 / (e - s).astype(jnp.float32)
    invb = jnp.broadcast_to(invlen[:, None], (_NREG, _LANES))

    result = _region_sc_kernel(table, i1, i2, i3, i4, invb)
    lengths = jnp.asarray(np.cumsum([_R] * _B).astype(np.int32))
    return (result, lengths)
